# Initial kernel scaffold; baseline (speedup 1.0000x reference)
import jax
import jax.numpy as jnp
from jax.experimental import pallas as pl


def kernel(embedding):
    raise NotImplementedError("write your pallas kernel here")


# speedup vs baseline: 1.4521x; 1.4521x over previous
import functools

import jax
import jax.numpy as jnp
from jax import lax
from jax.experimental import pallas as pl
from jax.experimental.pallas import tpu as pltpu
from jax.experimental.pallas import tpu_sc as plsc

NUM_ROWS = 1_000_000
DIM = 32
TOTAL = NUM_ROWS * DIM
NUM_CORES = 2
NUM_SUBCORES = 16
NUM_WORKERS = NUM_CORES * NUM_SUBCORES
PER_WORKER = TOTAL // NUM_WORKERS

CHUNK = 50_000
NBUF = 2
NCHUNKS = PER_WORKER // CHUNK

_MESH = plsc.VectorSubcoreMesh(core_axis_name="c", subcore_axis_name="s")


@functools.partial(
    pl.kernel,
    mesh=_MESH,
    out_type=jax.ShapeDtypeStruct((TOTAL,), jnp.float32),
    scratch_types=[
        pltpu.VMEM((CHUNK,), jnp.float32),
        pltpu.VMEM((CHUNK,), jnp.float32),
        pltpu.SemaphoreType.DMA((NBUF,)),
        pltpu.SemaphoreType.DMA((NBUF,)),
    ],
)
def _copy_kernel(in_hbm, out_hbm, buf0, buf1, in_sems, out_sems):
    wid = lax.axis_index("s") * NUM_CORES + lax.axis_index("c")
    base = wid * PER_WORKER
    bufs = (buf0, buf1)

    def copy_in(i):
        return pltpu.make_async_copy(
            in_hbm.at[pl.ds(base + i * CHUNK, CHUNK)],
            bufs[i % NBUF],
            in_sems.at[i % NBUF],
        )

    def copy_out(i):
        return pltpu.make_async_copy(
            bufs[i % NBUF],
            out_hbm.at[pl.ds(base + i * CHUNK, CHUNK)],
            out_sems.at[i % NBUF],
        )

    for j in range(NBUF):
        copy_in(j).start()
    for i in range(NCHUNKS):
        copy_in(i).wait()
        copy_out(i).start()
        if i + NBUF < NCHUNKS:
            copy_out(i).wait()
            copy_in(i + NBUF).start()
    for i in range(max(0, NCHUNKS - NBUF), NCHUNKS):
        copy_out(i).wait()


def kernel(embedding):
    flat = embedding.reshape(TOTAL)
    return _copy_kernel(flat).reshape(NUM_ROWS, DIM)


# speedup vs baseline: 1.6787x; 1.1560x over previous
import functools

import jax
import jax.numpy as jnp
from jax import lax
from jax.experimental import pallas as pl
from jax.experimental.pallas import tpu as pltpu
from jax.experimental.pallas import tpu_sc as plsc

NUM_ROWS = 1_000_000
DIM = 32
NUM_CORES = 2
NUM_SUBCORES = 16
NUM_WORKERS = NUM_CORES * NUM_SUBCORES

GROUPS = NUM_ROWS // 8
BASE_GROUPS = GROUPS // NUM_WORKERS
EXTRA = GROUPS - BASE_GROUPS * NUM_WORKERS
ROWS_MAIN = BASE_GROUPS * 8

CHUNK_ROWS = 504
NBUF = 2
NCHUNKS = ROWS_MAIN // CHUNK_ROWS

_MESH = plsc.VectorSubcoreMesh(core_axis_name="c", subcore_axis_name="s")


@functools.partial(
    pl.kernel,
    mesh=_MESH,
    out_type=jax.ShapeDtypeStruct((NUM_ROWS, DIM), jnp.float32),
    scratch_types=[
        pltpu.VMEM((CHUNK_ROWS, DIM), jnp.float32),
        pltpu.VMEM((CHUNK_ROWS, DIM), jnp.float32),
        pltpu.VMEM((8, DIM), jnp.float32),
        pltpu.SemaphoreType.DMA((NBUF,)),
        pltpu.SemaphoreType.DMA((NBUF,)),
        pltpu.SemaphoreType.DMA,
    ],
)
def _copy_kernel(in_hbm, out_hbm, buf0, buf1, tail_buf, in_sems, out_sems, tail_sem):
    wid = lax.axis_index("s") * NUM_CORES + lax.axis_index("c")
    base = pl.multiple_of(
        wid * ROWS_MAIN + jnp.minimum(wid, EXTRA) * 8, 8
    )
    bufs = (buf0, buf1)

    def copy_in(i):
        return pltpu.make_async_copy(
            in_hbm.at[pl.ds(pl.multiple_of(base + i * CHUNK_ROWS, 8), CHUNK_ROWS)],
            bufs[i % NBUF],
            in_sems.at[i % NBUF],
        )

    def copy_out(i):
        return pltpu.make_async_copy(
            bufs[i % NBUF],
            out_hbm.at[pl.ds(pl.multiple_of(base + i * CHUNK_ROWS, 8), CHUNK_ROWS)],
            out_sems.at[i % NBUF],
        )

    for j in range(NBUF):
        copy_in(j).start()
    for i in range(NCHUNKS):
        copy_in(i).wait()
        copy_out(i).start()
        if i + NBUF < NCHUNKS:
            copy_out(i).wait()
            copy_in(i + NBUF).start()

    @pl.when(wid < EXTRA)
    def _tail():
        tstart = pl.multiple_of(base + ROWS_MAIN, 8)
        pltpu.make_async_copy(
            in_hbm.at[pl.ds(tstart, 8)], tail_buf, tail_sem
        ).start()
        pltpu.make_async_copy(
            in_hbm.at[pl.ds(tstart, 8)], tail_buf, tail_sem
        ).wait()
        pltpu.make_async_copy(
            tail_buf, out_hbm.at[pl.ds(tstart, 8)], tail_sem
        ).start()
        pltpu.make_async_copy(
            tail_buf, out_hbm.at[pl.ds(tstart, 8)], tail_sem
        ).wait()

    for i in range(max(0, NCHUNKS - NBUF), NCHUNKS):
        copy_out(i).wait()


def kernel(embedding):
    return _copy_kernel(embedding)


# speedup vs baseline: 14.2604x; 8.4952x over previous
import functools

import jax
import jax.numpy as jnp
from jax import lax
from jax.experimental import pallas as pl
from jax.experimental.pallas import tpu as pltpu
from jax.experimental.pallas import tpu_sc as plsc

NUM_ROWS = 1_000_000
DIM = 32
NUM_CORES = 2
NUM_SUBCORES = 16
NUM_WORKERS = NUM_CORES * NUM_SUBCORES

CHUNK_COLS = 1280
NFULL = NUM_ROWS // CHUNK_COLS
TAIL_COLS = NUM_ROWS - NFULL * CHUNK_COLS
BASE_CHUNKS = NFULL // NUM_WORKERS
EXTRA_CHUNKS = NFULL - BASE_CHUNKS * NUM_WORKERS
TAIL_WORKER = EXTRA_CHUNKS
NBUF = 2

_MESH = plsc.VectorSubcoreMesh(core_axis_name="c", subcore_axis_name="s")


@functools.partial(
    pl.kernel,
    mesh=_MESH,
    out_type=jax.ShapeDtypeStruct((DIM, NUM_ROWS), jnp.float32),
    scratch_types=[
        pltpu.VMEM((DIM, CHUNK_COLS), jnp.float32),
        pltpu.VMEM((DIM, CHUNK_COLS), jnp.float32),
        pltpu.VMEM((DIM, 256), jnp.float32),
        pltpu.VMEM((DIM, 128), jnp.float32),
        pltpu.SemaphoreType.DMA((NBUF,)),
        pltpu.SemaphoreType.DMA((NBUF,)),
        pltpu.SemaphoreType.DMA,
    ],
)
def _copy_kernel(
    in_hbm, out_hbm, buf0, buf1, tail_buf, tail2_buf, in_sems, out_sems, tail_sem
):
    wid = lax.axis_index("s") * NUM_CORES + lax.axis_index("c")
    bufs = (buf0, buf1)

    def col_start(k):
        return pl.multiple_of((wid + k * NUM_WORKERS) * CHUNK_COLS, 128)

    def copy_in(k):
        return pltpu.make_async_copy(
            in_hbm.at[:, pl.ds(col_start(k), CHUNK_COLS)],
            bufs[k % NBUF],
            in_sems.at[k % NBUF],
        )

    def copy_out(k):
        return pltpu.make_async_copy(
            bufs[k % NBUF],
            out_hbm.at[:, pl.ds(col_start(k), CHUNK_COLS)],
            out_sems.at[k % NBUF],
        )

    for j in range(NBUF):
        copy_in(j).start()
    for k in range(BASE_CHUNKS):
        copy_in(k).wait()
        copy_out(k).start()
        if k + NBUF < BASE_CHUNKS:
            copy_out(k).wait()
            copy_in(k + NBUF).start()

    for k in range(max(0, BASE_CHUNKS - NBUF), BASE_CHUNKS):
        copy_out(k).wait()

    @pl.when(wid < EXTRA_CHUNKS)
    def _extra():
        start = pl.multiple_of((BASE_CHUNKS * NUM_WORKERS + wid) * CHUNK_COLS, 128)
        pltpu.make_async_copy(
            in_hbm.at[:, pl.ds(start, CHUNK_COLS)], buf0, tail_sem
        ).start()
        pltpu.make_async_copy(
            in_hbm.at[:, pl.ds(start, CHUNK_COLS)], buf0, tail_sem
        ).wait()
        pltpu.make_async_copy(
            buf0, out_hbm.at[:, pl.ds(start, CHUNK_COLS)], tail_sem
        ).start()
        pltpu.make_async_copy(
            buf0, out_hbm.at[:, pl.ds(start, CHUNK_COLS)], tail_sem
        ).wait()

    @pl.when(wid == TAIL_WORKER)
    def _tail():
        start = pl.multiple_of(NFULL * CHUNK_COLS, 128)
        pltpu.make_async_copy(
            in_hbm.at[:, pl.ds(start, 256)], tail_buf, tail_sem
        ).start()
        pltpu.make_async_copy(
            in_hbm.at[:, pl.ds(start, 256)], tail_buf, tail_sem
        ).wait()
        pltpu.make_async_copy(
            tail_buf, out_hbm.at[:, pl.ds(start, 256)], tail_sem
        ).start()
        pltpu.make_async_copy(
            tail_buf, out_hbm.at[:, pl.ds(start, 256)], tail_sem
        ).wait()
        last = pl.multiple_of(NFULL * CHUNK_COLS + 256 + wid * 0, 128)
        pltpu.make_async_copy(
            in_hbm.at[:, pl.ds(last, 128)], tail2_buf, tail_sem
        ).start()
        pltpu.make_async_copy(
            in_hbm.at[:, pl.ds(last, 128)], tail2_buf, tail_sem
        ).wait()
        pltpu.make_async_copy(
            tail2_buf, out_hbm.at[:, pl.ds(last, 128)], tail_sem
        ).start()
        pltpu.make_async_copy(
            tail2_buf, out_hbm.at[:, pl.ds(last, 128)], tail_sem
        ).wait()


def kernel(embedding):
    return _copy_kernel(embedding.T).T


# speedup vs baseline: 14.3708x; 1.0077x over previous
import functools

import jax
import jax.numpy as jnp
from jax import lax
from jax.experimental import pallas as pl
from jax.experimental.pallas import tpu as pltpu
from jax.experimental.pallas import tpu_sc as plsc

NUM_ROWS = 1_000_000
DIM = 32
NUM_CORES = 2
NUM_SUBCORES = 16
NUM_WORKERS = NUM_CORES * NUM_SUBCORES

ALIGNED_COLS = (NUM_ROWS // 128) * 128
CHUNK_COLS = 1792
NFULL = ALIGNED_COLS // CHUNK_COLS
BASE_CHUNKS = NFULL // NUM_WORKERS
EXTRA_CHUNKS = NFULL - BASE_CHUNKS * NUM_WORKERS
TAIL_WORKER = EXTRA_CHUNKS
NBUF = 2

_MESH = plsc.VectorSubcoreMesh(core_axis_name="c", subcore_axis_name="s")


@functools.partial(
    pl.kernel,
    mesh=_MESH,
    out_type=jax.ShapeDtypeStruct((DIM, NUM_ROWS), jnp.float32),
    scratch_types=[
        pltpu.VMEM((DIM, CHUNK_COLS), jnp.float32),
        pltpu.VMEM((DIM, CHUNK_COLS), jnp.float32),
        pltpu.VMEM((DIM, 128), jnp.float32),
        pltpu.SemaphoreType.DMA((NBUF,)),
        pltpu.SemaphoreType.DMA((NBUF,)),
        pltpu.SemaphoreType.DMA,
    ],
)
def _copy_kernel(in_hbm, out_hbm, buf0, buf1, tail_buf, in_sems, out_sems, tail_sem):
    wid = lax.axis_index("s") * NUM_CORES + lax.axis_index("c")
    bufs = (buf0, buf1)

    def col_start(k):
        return pl.multiple_of((wid + k * NUM_WORKERS) * CHUNK_COLS, 128)

    def copy_in(k):
        return pltpu.make_async_copy(
            in_hbm.at[:, pl.ds(col_start(k), CHUNK_COLS)],
            bufs[k % NBUF],
            in_sems.at[k % NBUF],
        )

    def copy_out(k):
        return pltpu.make_async_copy(
            bufs[k % NBUF],
            out_hbm.at[:, pl.ds(col_start(k), CHUNK_COLS)],
            out_sems.at[k % NBUF],
        )

    for j in range(NBUF):
        copy_in(j).start()
    for k in range(BASE_CHUNKS):
        copy_in(k).wait()
        copy_out(k).start()
        if k + NBUF < BASE_CHUNKS:
            copy_out(k).wait()
            copy_in(k + NBUF).start()
    for k in range(max(0, BASE_CHUNKS - NBUF), BASE_CHUNKS):
        copy_out(k).wait()

    @pl.when(wid < EXTRA_CHUNKS)
    def _extra():
        start = pl.multiple_of((BASE_CHUNKS * NUM_WORKERS + wid) * CHUNK_COLS, 128)
        pltpu.make_async_copy(
            in_hbm.at[:, pl.ds(start, CHUNK_COLS)], buf0, tail_sem
        ).start()
        pltpu.make_async_copy(
            in_hbm.at[:, pl.ds(start, CHUNK_COLS)], buf0, tail_sem
        ).wait()
        pltpu.make_async_copy(
            buf0, out_hbm.at[:, pl.ds(start, CHUNK_COLS)], tail_sem
        ).start()
        pltpu.make_async_copy(
            buf0, out_hbm.at[:, pl.ds(start, CHUNK_COLS)], tail_sem
        ).wait()

    @pl.when(wid == TAIL_WORKER)
    def _tail():
        last = pl.multiple_of(ALIGNED_COLS + wid * 0, 128)
        pltpu.make_async_copy(
            in_hbm.at[:, pl.ds(last, 128)], tail_buf, tail_sem
        ).start()
        pltpu.make_async_copy(
            in_hbm.at[:, pl.ds(last, 128)], tail_buf, tail_sem
        ).wait()
        pltpu.make_async_copy(
            tail_buf, out_hbm.at[:, pl.ds(last, 128)], tail_sem
        ).start()
        pltpu.make_async_copy(
            tail_buf, out_hbm.at[:, pl.ds(last, 128)], tail_sem
        ).wait()


def kernel(embedding):
    return _copy_kernel(embedding.T).T
